# Initial kernel scaffold; baseline (speedup 1.0000x reference)
import jax
import jax.numpy as jnp
from jax.experimental import pallas as pl


def kernel(x, edge_index, edge_attr, u, W1, b1, W2, b2, Wn, bn, Wg, bg):
    raise NotImplementedError("write your pallas kernel here")


# speedup vs baseline: 4.3995x; 4.3995x over previous
import functools

import jax
import jax.numpy as jnp
from jax import lax
from jax.experimental import pallas as pl
from jax.experimental.pallas import tpu as pltpu
from jax.experimental.pallas import tpu_sc as plsc

N_NODES = 10000
N_EDGES = 320000
D = 128
DE = 16

_NC = 2
_NS = 16
_NW = _NC * _NS
_EPW = N_EDGES // _NW
_CH = 80
_NCHUNK = _EPW // _CH

_BE = 2560
_NBLK = N_EDGES // _BE


def _node_proj_body(x_ref, wa_ref, wb_ref, xa_ref, xb_ref):
    x = x_ref[...]
    xa_ref[...] = jnp.dot(x, wa_ref[...], preferred_element_type=jnp.float32)
    xb_ref[...] = jnp.dot(x, wb_ref[...], preferred_element_type=jnp.float32)


def _node_proj(x, w1a, w1b):
    return pl.pallas_call(
        _node_proj_body,
        out_shape=(
            jax.ShapeDtypeStruct((N_NODES, D), jnp.float32),
            jax.ShapeDtypeStruct((N_NODES, D), jnp.float32),
        ),
    )(x, w1a, w1b)


def _edge_gather(xa, xb, src, dst):
    mesh = plsc.VectorSubcoreMesh(core_axis_name="c", subcore_axis_name="s")

    @functools.partial(
        pl.kernel,
        mesh=mesh,
        out_type=(
            jax.ShapeDtypeStruct((N_EDGES, D), jnp.float32),
            jax.ShapeDtypeStruct((N_EDGES, D), jnp.float32),
        ),
        scratch_types=[
            pltpu.VMEM((_CH,), jnp.int32),
            pltpu.VMEM((_CH,), jnp.int32),
            pltpu.VMEM((_CH, D), jnp.float32),
            pltpu.VMEM((_CH, D), jnp.float32),
            pltpu.SemaphoreType.DMA,
            pltpu.SemaphoreType.DMA,
        ],
    )
    def gather_k(xa_hbm, xb_hbm, src_hbm, dst_hbm, ga_hbm, gb_hbm,
                 si_v, di_v, ra_v, rb_v, sem_a, sem_b):
        wid = lax.axis_index("s") * _NC + lax.axis_index("c")

        def body(c, carry):
            base = wid * _EPW + c * _CH
            pltpu.sync_copy(src_hbm.at[pl.ds(base, _CH)], si_v)
            pltpu.sync_copy(dst_hbm.at[pl.ds(base, _CH)], di_v)
            cp_a = pltpu.async_copy(xa_hbm.at[si_v], ra_v, sem_a)
            cp_b = pltpu.async_copy(xb_hbm.at[di_v], rb_v, sem_b)
            cp_a.wait()
            cp_b.wait()
            pltpu.sync_copy(ra_v, ga_hbm.at[pl.ds(base, _CH)])
            pltpu.sync_copy(rb_v, gb_hbm.at[pl.ds(base, _CH)])
            return carry

        lax.fori_loop(0, _NCHUNK, body, 0)

    return gather_k(xa, xb, src, dst)


def _edge_mlp_body(ga_ref, gb_ref, ea_ref, w1c_ref, b1_ref, w2_ref, b2_ref,
                   wn_ref, bn_ref, wg_ref, bg_ref, u_ref, out_ref, acc_ref):
    step = pl.program_id(0)

    @pl.when(step == 0)
    def _():
        acc_ref[...] = jnp.zeros_like(acc_ref)

    m1 = ga_ref[...] + gb_ref[...] + b1_ref[...]
    m1 += jnp.dot(ea_ref[...], w1c_ref[...], preferred_element_type=jnp.float32)
    m1 = jnp.maximum(m1, 0.0)
    m = jnp.dot(m1, w2_ref[...], preferred_element_type=jnp.float32)
    m = jnp.maximum(m + b2_ref[...], 0.0)
    acc_ref[...] += jnp.sum(m, axis=0, keepdims=True)

    @pl.when(step == _NBLK - 1)
    def _():
        s = acc_ref[...]
        snf = jnp.dot(s, wn_ref[...], preferred_element_type=jnp.float32)
        snf += jnp.float32(N_NODES) * bn_ref[...]
        g = jnp.dot(u_ref[...], wg_ref[:D, :], preferred_element_type=jnp.float32)
        g += jnp.dot(snf, wg_ref[D:, :], preferred_element_type=jnp.float32)
        out_ref[...] = jnp.maximum(g + bg_ref[...], 0.0)


def _edge_mlp(ga, gb, ea, w1c, b1, w2, b2, wn, bn, wg, bg, u):
    fixed = lambda i: (0, 0)
    return pl.pallas_call(
        _edge_mlp_body,
        grid=(_NBLK,),
        in_specs=[
            pl.BlockSpec((_BE, D), lambda i: (i, 0)),
            pl.BlockSpec((_BE, D), lambda i: (i, 0)),
            pl.BlockSpec((_BE, DE), lambda i: (i, 0)),
            pl.BlockSpec((DE, D), fixed),
            pl.BlockSpec((1, D), fixed),
            pl.BlockSpec((D, D), fixed),
            pl.BlockSpec((1, D), fixed),
            pl.BlockSpec((D, D), fixed),
            pl.BlockSpec((1, D), fixed),
            pl.BlockSpec((2 * D, D), fixed),
            pl.BlockSpec((1, D), fixed),
            pl.BlockSpec((1, D), fixed),
        ],
        out_specs=pl.BlockSpec((1, D), fixed),
        out_shape=jax.ShapeDtypeStruct((1, D), jnp.float32),
        scratch_shapes=[pltpu.VMEM((1, D), jnp.float32)],
    )(ga, gb, ea, w1c, b1, w2, b2, wn, bn, wg, bg, u)


def kernel(x, edge_index, edge_attr, u, W1, b1, W2, b2, Wn, bn, Wg, bg):
    src = edge_index[0]
    dst = edge_index[1]
    w1a, w1b, w1c = W1[:D], W1[D:2 * D], W1[2 * D:]
    xa, xb = _node_proj(x, w1a, w1b)
    ga, gb = _edge_gather(xa, xb, src, dst)
    return _edge_mlp(ga, gb, edge_attr, w1c, b1.reshape(1, D), W2,
                     b2.reshape(1, D), Wn, bn.reshape(1, D), Wg,
                     bg.reshape(1, D), u)


# speedup vs baseline: 5.4205x; 1.2321x over previous
import functools

import jax
import jax.numpy as jnp
from jax import lax
from jax.experimental import pallas as pl
from jax.experimental.pallas import tpu as pltpu
from jax.experimental.pallas import tpu_sc as plsc

N_NODES = 10000
N_EDGES = 320000
D = 128
DE = 16

_NC = 2
_NS = 16
_NW = _NC * _NS
_EPW = N_EDGES // _NW
_CH = 40
_NCHUNK = _EPW // _CH

_BE = 2560
_NBLK = N_EDGES // _BE


def _node_proj_body(x_ref, wa_ref, wb_ref, xa_ref, xb_ref):
    x = x_ref[...]
    xa_ref[...] = jnp.dot(x, wa_ref[...], preferred_element_type=jnp.float32)
    xb_ref[...] = jnp.dot(x, wb_ref[...], preferred_element_type=jnp.float32)


def _node_proj(x, w1a, w1b):
    return pl.pallas_call(
        _node_proj_body,
        out_shape=(
            jax.ShapeDtypeStruct((N_NODES, D), jnp.float32),
            jax.ShapeDtypeStruct((N_NODES, D), jnp.float32),
        ),
    )(x, w1a, w1b)


def _edge_gather(xa, xb, src, dst):
    mesh = plsc.VectorSubcoreMesh(core_axis_name="c", subcore_axis_name="s")

    @functools.partial(
        pl.kernel,
        mesh=mesh,
        out_type=(
            jax.ShapeDtypeStruct((N_EDGES, D), jnp.float32),
            jax.ShapeDtypeStruct((N_EDGES, D), jnp.float32),
        ),
        scratch_types=[
            pltpu.VMEM((_CH,), jnp.int32),
            pltpu.VMEM((_CH,), jnp.int32),
            pltpu.VMEM((_CH,), jnp.int32),
            pltpu.VMEM((_CH,), jnp.int32),
            pltpu.VMEM((_CH, D), jnp.float32),
            pltpu.VMEM((_CH, D), jnp.float32),
            pltpu.VMEM((_CH, D), jnp.float32),
            pltpu.VMEM((_CH, D), jnp.float32),
            pltpu.SemaphoreType.DMA,
            pltpu.SemaphoreType.DMA,
            pltpu.SemaphoreType.DMA,
            pltpu.SemaphoreType.DMA,
            pltpu.SemaphoreType.DMA,
            pltpu.SemaphoreType.DMA,
        ],
    )
    def gather_k(xa_hbm, xb_hbm, src_hbm, dst_hbm, ga_hbm, gb_hbm,
                 si0, di0, si1, di1, ra0, rb0, ra1, rb1,
                 ixs0, ixs1, gs0, gs1, sts0, sts1):
        wid = lax.axis_index("s") * _NC + lax.axis_index("c")
        SI, DI = (si0, si1), (di0, di1)
        RA, RB = (ra0, ra1), (rb0, rb1)
        IXS, GS, STS = (ixs0, ixs1), (gs0, gs1), (sts0, sts1)

        def idx_copies(g, b):
            base = wid * _EPW + g * _CH
            return (
                pltpu.make_async_copy(src_hbm.at[pl.ds(base, _CH)], SI[b], IXS[b]),
                pltpu.make_async_copy(dst_hbm.at[pl.ds(base, _CH)], DI[b], IXS[b]),
            )

        def gath_copies(b):
            return (
                pltpu.make_async_copy(xa_hbm.at[SI[b]], RA[b], GS[b]),
                pltpu.make_async_copy(xb_hbm.at[DI[b]], RB[b], GS[b]),
            )

        def store_copies(g, b):
            base = wid * _EPW + g * _CH
            return (
                pltpu.make_async_copy(RA[b], ga_hbm.at[pl.ds(base, _CH)], STS[b]),
                pltpu.make_async_copy(RB[b], gb_hbm.at[pl.ds(base, _CH)], STS[b]),
            )

        def fire(cs):
            for c in cs:
                c.start()

        def drain(cs):
            for c in cs:
                c.wait()

        fire(idx_copies(0, 0))
        drain(idx_copies(0, 0))
        fire(gath_copies(0))
        fire(idx_copies(1, 1))

        def body(c, carry):
            for b in (0, 1):
                g = 2 * c + b
                o = 1 - b
                @pl.when(g >= 1)
                def _(g=g, o=o):
                    drain(store_copies(g - 1, o))

                @pl.when(g <= _NCHUNK - 2)
                def _(g=g, o=o):
                    drain(idx_copies(g + 1, o))
                    fire(gath_copies(o))

                drain(gath_copies(b))
                fire(store_copies(g, b))

                @pl.when(g <= _NCHUNK - 3)
                def _(g=g, b=b):
                    fire(idx_copies(g + 2, b))
            return carry

        lax.fori_loop(0, _NCHUNK // 2, body, 0)
        drain(store_copies(_NCHUNK - 1, 1))

    return gather_k(xa, xb, src, dst)


def _edge_mlp_body(ga_ref, gb_ref, ea_ref, w1c_ref, b1_ref, w2_ref, b2_ref,
                   wn_ref, bn_ref, wg_ref, bg_ref, u_ref, out_ref, acc_ref):
    step = pl.program_id(0)

    @pl.when(step == 0)
    def _():
        acc_ref[...] = jnp.zeros_like(acc_ref)

    m1 = ga_ref[...] + gb_ref[...] + b1_ref[...]
    m1 += jnp.dot(ea_ref[...], w1c_ref[...], preferred_element_type=jnp.float32)
    m1 = jnp.maximum(m1, 0.0).astype(jnp.bfloat16)
    m = jnp.dot(m1, w2_ref[...], preferred_element_type=jnp.float32)
    m = jnp.maximum(m + b2_ref[...], 0.0)
    acc_ref[...] += jnp.sum(m, axis=0, keepdims=True)

    @pl.when(step == _NBLK - 1)
    def _():
        s = acc_ref[...]
        snf = jnp.dot(s, wn_ref[...], preferred_element_type=jnp.float32)
        snf += jnp.float32(N_NODES) * bn_ref[...]
        g = jnp.dot(u_ref[...], wg_ref[:D, :], preferred_element_type=jnp.float32)
        g += jnp.dot(snf, wg_ref[D:, :], preferred_element_type=jnp.float32)
        out_ref[...] = jnp.maximum(g + bg_ref[...], 0.0)


def _edge_mlp(ga, gb, ea, w1c, b1, w2, b2, wn, bn, wg, bg, u):
    fixed = lambda i: (0, 0)
    return pl.pallas_call(
        _edge_mlp_body,
        grid=(_NBLK,),
        in_specs=[
            pl.BlockSpec((_BE, D), lambda i: (i, 0)),
            pl.BlockSpec((_BE, D), lambda i: (i, 0)),
            pl.BlockSpec((_BE, DE), lambda i: (i, 0)),
            pl.BlockSpec((DE, D), fixed),
            pl.BlockSpec((1, D), fixed),
            pl.BlockSpec((D, D), fixed),
            pl.BlockSpec((1, D), fixed),
            pl.BlockSpec((D, D), fixed),
            pl.BlockSpec((1, D), fixed),
            pl.BlockSpec((2 * D, D), fixed),
            pl.BlockSpec((1, D), fixed),
            pl.BlockSpec((1, D), fixed),
        ],
        out_specs=pl.BlockSpec((1, D), fixed),
        out_shape=jax.ShapeDtypeStruct((1, D), jnp.float32),
        scratch_shapes=[pltpu.VMEM((1, D), jnp.float32)],
    )(ga, gb, ea, w1c, b1, w2, b2, wn, bn, wg, bg, u)


def kernel(x, edge_index, edge_attr, u, W1, b1, W2, b2, Wn, bn, Wg, bg):
    src = edge_index[0]
    dst = edge_index[1]
    w1a, w1b, w1c = W1[:D], W1[D:2 * D], W1[2 * D:]
    xa, xb = _node_proj(x, w1a, w1b)
    ga, gb = _edge_gather(xa, xb, src, dst)
    return _edge_mlp(ga, gb, edge_attr, w1c, b1.reshape(1, D),
                     W2.astype(jnp.bfloat16), b2.reshape(1, D), Wn,
                     bn.reshape(1, D), Wg, bg.reshape(1, D), u)


# speedup vs baseline: 5.7406x; 1.0590x over previous
import functools

import jax
import jax.numpy as jnp
from jax import lax
from jax.experimental import pallas as pl
from jax.experimental.pallas import tpu as pltpu
from jax.experimental.pallas import tpu_sc as plsc

N_NODES = 10000
N_EDGES = 320000
D = 128
DE = 16

_NPART = 5
_EP = N_EDGES // _NPART

_NC = 2
_NS = 16
_NW = _NC * _NS
_EPW = _EP // _NW
_CH = 40
_NCHUNK = _EPW // _CH

_BE = 2560
_NBLK = _EP // _BE


def _node_proj_body(x_ref, wa_ref, wb_ref, xa_ref, xb_ref):
    x = x_ref[...]
    xa_ref[...] = jnp.dot(x, wa_ref[...], preferred_element_type=jnp.float32)
    xb_ref[...] = jnp.dot(x, wb_ref[...], preferred_element_type=jnp.float32)


def _node_proj(x, w1a, w1b):
    return pl.pallas_call(
        _node_proj_body,
        out_shape=(
            jax.ShapeDtypeStruct((N_NODES, D), jnp.float32),
            jax.ShapeDtypeStruct((N_NODES, D), jnp.float32),
        ),
    )(x, w1a, w1b)


def _edge_gather(xa, xb, src, dst):
    mesh = plsc.VectorSubcoreMesh(core_axis_name="c", subcore_axis_name="s")

    @functools.partial(
        pl.kernel,
        mesh=mesh,
        out_type=(
            jax.ShapeDtypeStruct((_EP, D), jnp.float32),
            jax.ShapeDtypeStruct((_EP, D), jnp.float32),
        ),
        scratch_types=[
            pltpu.VMEM((_CH,), jnp.int32),
            pltpu.VMEM((_CH,), jnp.int32),
            pltpu.VMEM((_CH,), jnp.int32),
            pltpu.VMEM((_CH,), jnp.int32),
            pltpu.VMEM((_CH, D), jnp.float32),
            pltpu.VMEM((_CH, D), jnp.float32),
            pltpu.VMEM((_CH, D), jnp.float32),
            pltpu.VMEM((_CH, D), jnp.float32),
            pltpu.SemaphoreType.DMA,
            pltpu.SemaphoreType.DMA,
            pltpu.SemaphoreType.DMA,
            pltpu.SemaphoreType.DMA,
            pltpu.SemaphoreType.DMA,
            pltpu.SemaphoreType.DMA,
        ],
    )
    def gather_k(xa_hbm, xb_hbm, src_hbm, dst_hbm, ga_hbm, gb_hbm,
                 si0, di0, si1, di1, ra0, rb0, ra1, rb1,
                 ixs0, ixs1, gs0, gs1, sts0, sts1):
        wid = lax.axis_index("s") * _NC + lax.axis_index("c")
        SI, DI = (si0, si1), (di0, di1)
        RA, RB = (ra0, ra1), (rb0, rb1)
        IXS, GS, STS = (ixs0, ixs1), (gs0, gs1), (sts0, sts1)

        def idx_copies(g, b):
            base = wid * _EPW + g * _CH
            return (
                pltpu.make_async_copy(src_hbm.at[pl.ds(base, _CH)], SI[b], IXS[b]),
                pltpu.make_async_copy(dst_hbm.at[pl.ds(base, _CH)], DI[b], IXS[b]),
            )

        def gath_copies(b):
            return (
                pltpu.make_async_copy(xa_hbm.at[SI[b]], RA[b], GS[b]),
                pltpu.make_async_copy(xb_hbm.at[DI[b]], RB[b], GS[b]),
            )

        def store_copies(g, b):
            base = wid * _EPW + g * _CH
            return (
                pltpu.make_async_copy(RA[b], ga_hbm.at[pl.ds(base, _CH)], STS[b]),
                pltpu.make_async_copy(RB[b], gb_hbm.at[pl.ds(base, _CH)], STS[b]),
            )

        def fire(cs):
            for c in cs:
                c.start()

        def drain(cs):
            for c in cs:
                c.wait()

        fire(idx_copies(0, 0))
        drain(idx_copies(0, 0))
        fire(gath_copies(0))
        fire(idx_copies(1, 1))

        def body(c, carry):
            for b in (0, 1):
                g = 2 * c + b
                o = 1 - b
                @pl.when(g >= 1)
                def _(g=g, o=o):
                    drain(store_copies(g - 1, o))

                @pl.when(g <= _NCHUNK - 2)
                def _(g=g, o=o):
                    drain(idx_copies(g + 1, o))
                    fire(gath_copies(o))

                drain(gath_copies(b))
                fire(store_copies(g, b))

                @pl.when(g <= _NCHUNK - 3)
                def _(g=g, b=b):
                    fire(idx_copies(g + 2, b))
            return carry

        lax.fori_loop(0, _NCHUNK // 2, body, 0)
        drain(store_copies(_NCHUNK - 1, 1))

    return gather_k(xa, xb, src, dst)


def _edge_partial_body(ga_ref, gb_ref, ea_ref, w1c_ref, b1_ref, w2_ref,
                       b2_ref, out_ref, acc_ref):
    step = pl.program_id(0)

    @pl.when(step == 0)
    def _():
        acc_ref[...] = jnp.zeros_like(acc_ref)

    m1 = ga_ref[...] + gb_ref[...] + b1_ref[...]
    m1 += jnp.dot(ea_ref[...], w1c_ref[...], preferred_element_type=jnp.float32)
    m1 = jnp.maximum(m1, 0.0).astype(jnp.bfloat16)
    m = jnp.dot(m1, w2_ref[...], preferred_element_type=jnp.float32)
    m = jnp.maximum(m + b2_ref[...], 0.0)
    acc_ref[...] += jnp.sum(m, axis=0, keepdims=True)

    @pl.when(step == _NBLK - 1)
    def _():
        out_ref[...] = acc_ref[...]


def _edge_partial(ga, gb, ea, w1c, b1, w2, b2):
    fixed = lambda i: (0, 0)
    return pl.pallas_call(
        _edge_partial_body,
        grid=(_NBLK,),
        in_specs=[
            pl.BlockSpec((_BE, D), lambda i: (i, 0)),
            pl.BlockSpec((_BE, D), lambda i: (i, 0)),
            pl.BlockSpec((_BE, DE), lambda i: (i, 0)),
            pl.BlockSpec((DE, D), fixed),
            pl.BlockSpec((1, D), fixed),
            pl.BlockSpec((D, D), fixed),
            pl.BlockSpec((1, D), fixed),
        ],
        out_specs=pl.BlockSpec((1, D), fixed),
        out_shape=jax.ShapeDtypeStruct((1, D), jnp.float32),
        scratch_shapes=[pltpu.VMEM((1, D), jnp.float32)],
    )(ga, gb, ea, w1c, b1, w2, b2)


def _final_body(sp_ref, wn_ref, bn_ref, wg_ref, bg_ref, u_ref, out_ref):
    s = jnp.sum(sp_ref[...], axis=0, keepdims=True)
    snf = jnp.dot(s, wn_ref[...], preferred_element_type=jnp.float32)
    snf += jnp.float32(N_NODES) * bn_ref[...]
    g = jnp.dot(u_ref[...], wg_ref[:D, :], preferred_element_type=jnp.float32)
    g += jnp.dot(snf, wg_ref[D:, :], preferred_element_type=jnp.float32)
    out_ref[...] = jnp.maximum(g + bg_ref[...], 0.0)


def _final(sp, wn, bn, wg, bg, u):
    return pl.pallas_call(
        _final_body,
        out_shape=jax.ShapeDtypeStruct((1, D), jnp.float32),
    )(sp, wn, bn, wg, bg, u)


def kernel(x, edge_index, edge_attr, u, W1, b1, W2, b2, Wn, bn, Wg, bg):
    src = edge_index[0]
    dst = edge_index[1]
    w1a, w1b, w1c = W1[:D], W1[D:2 * D], W1[2 * D:]
    xa, xb = _node_proj(x, w1a, w1b)
    b1r = b1.reshape(1, D)
    b2r = b2.reshape(1, D)
    w2b = W2.astype(jnp.bfloat16)
    parts = []
    for p in range(_NPART):
        lo, hi = p * _EP, (p + 1) * _EP
        ga, gb = _edge_gather(xa, xb, src[lo:hi], dst[lo:hi])
        parts.append(_edge_partial(ga, gb, edge_attr[lo:hi], w1c, b1r,
                                   w2b, b2r))
    sp = jnp.concatenate(parts + [jnp.zeros((8 - _NPART, D), jnp.float32)],
                         axis=0)
    return _final(sp, Wn, bn.reshape(1, D), Wg, bg.reshape(1, D), u)
